# speedup vs baseline: 1.1026x; 1.1026x over previous
import functools

import jax
import jax.numpy as jnp
from jax import lax
from jax.experimental import pallas as pl
from jax.experimental.pallas import tpu as pltpu
from jax.experimental.pallas import tpu_sc as plsc

BATCH = 16384
NUM_CORES = 2
NUM_SUBCORES = 16
NUM_WORKERS = NUM_CORES * NUM_SUBCORES
B_PER_W = BATCH // NUM_WORKERS
CHUNK = 128
NCHUNK = B_PER_W // CHUNK


def _build():
    mesh = plsc.VectorSubcoreMesh(core_axis_name="c", subcore_axis_name="s")

    @functools.partial(
        pl.kernel,
        mesh=mesh,
        out_type=jax.ShapeDtypeStruct((BATCH,), jnp.float32),
        scratch_types=[
            pltpu.VMEM((B_PER_W,), jnp.int32),
            pltpu.VMEM((B_PER_W,), jnp.float32),
            pltpu.SemaphoreType.DMA,
        ],
    )
    def gather_kernel(table_hbm, idx_hbm, out_hbm, idx_v, rows_v, sem):
        wid = lax.axis_index("s") * NUM_CORES + lax.axis_index("c")
        base = wid * B_PER_W
        pltpu.sync_copy(idx_hbm.at[pl.ds(base, B_PER_W)], idx_v)
        copies = []
        for j in range(NCHUNK):
            sl = pl.ds(j * CHUNK, CHUNK)
            copies.append(
                pltpu.async_copy(table_hbm.at[idx_v.at[sl]], rows_v.at[sl], sem)
            )
        for c in copies:
            c.wait()
        pltpu.sync_copy(rows_v, out_hbm.at[pl.ds(base, B_PER_W)])

    return gather_kernel


_gather = _build()


@jax.jit
def kernel(data, ratio_target_lookup, targets):
    del data
    return _gather(ratio_target_lookup, targets)


# speedup vs baseline: 1.1110x; 1.0076x over previous
import functools

import jax
import jax.numpy as jnp
from jax import lax
from jax.experimental import pallas as pl
from jax.experimental.pallas import tpu as pltpu
from jax.experimental.pallas import tpu_sc as plsc

BATCH = 16384
NUM_CORES = 2
NUM_SUBCORES = 16
NUM_WORKERS = NUM_CORES * NUM_SUBCORES
B_PER_W = BATCH // NUM_WORKERS
CHUNK = 128
NCHUNK = B_PER_W // CHUNK


def _build():
    mesh = plsc.VectorSubcoreMesh(core_axis_name="c", subcore_axis_name="s")

    @functools.partial(
        pl.kernel,
        mesh=mesh,
        out_type=jax.ShapeDtypeStruct((BATCH,), jnp.float32),
        scratch_types=[
            pltpu.VMEM((B_PER_W,), jnp.int32),
            pltpu.VMEM((B_PER_W,), jnp.float32),
            pltpu.SemaphoreType.DMA,
            pltpu.SemaphoreType.DMA,
            pltpu.SemaphoreType.DMA,
        ],
    )
    def gather_kernel(table_hbm, idx_hbm, out_hbm, idx_v, rows_v,
                      sem_i, sem_g, sem_o):
        wid = lax.axis_index("s") * NUM_CORES + lax.axis_index("c")
        base = wid * B_PER_W
        idx_copies = []
        for j in range(NCHUNK):
            sl = pl.ds(j * CHUNK, CHUNK)
            idx_copies.append(
                pltpu.async_copy(idx_hbm.at[pl.ds(base + j * CHUNK, CHUNK)],
                                 idx_v.at[sl], sem_i)
            )
        gathers = []
        for j in range(NCHUNK):
            sl = pl.ds(j * CHUNK, CHUNK)
            idx_copies[j].wait()
            gathers.append(
                pltpu.async_copy(table_hbm.at[idx_v.at[sl]], rows_v.at[sl],
                                 sem_g)
            )
        out_copies = []
        for j in range(NCHUNK):
            sl = pl.ds(j * CHUNK, CHUNK)
            gathers[j].wait()
            out_copies.append(
                pltpu.async_copy(rows_v.at[sl],
                                 out_hbm.at[pl.ds(base + j * CHUNK, CHUNK)],
                                 sem_o)
            )
        for c in out_copies:
            c.wait()

    return gather_kernel


_gather = _build()


@jax.jit
def kernel(data, ratio_target_lookup, targets):
    del data
    return _gather(ratio_target_lookup, targets)


# speedup vs baseline: 1.1131x; 1.0020x over previous
import functools

import jax
import jax.numpy as jnp
from jax import lax
from jax.experimental import pallas as pl
from jax.experimental.pallas import tpu as pltpu
from jax.experimental.pallas import tpu_sc as plsc

BATCH = 16384
NUM_CORES = 2
NUM_SUBCORES = 16
NUM_WORKERS = NUM_CORES * NUM_SUBCORES
B_PER_W = BATCH // NUM_WORKERS
CHUNK = 128
NCHUNK = B_PER_W // CHUNK


def _build():
    mesh = plsc.VectorSubcoreMesh(core_axis_name="c", subcore_axis_name="s")

    @functools.partial(
        pl.kernel,
        mesh=mesh,
        out_type=jax.ShapeDtypeStruct((BATCH,), jnp.float32),
        scratch_types=[
            pltpu.VMEM((B_PER_W,), jnp.int32),
            pltpu.VMEM((B_PER_W,), jnp.float32),
            pltpu.SemaphoreType.DMA,
            pltpu.SemaphoreType.DMA,
            pltpu.SemaphoreType.DMA,
        ],
    )
    def gather_kernel(table_hbm, idx_hbm, out_hbm, idx_v, rows_v,
                      sem_i, sem_g, sem_o):
        wid = lax.axis_index("s") * NUM_CORES + lax.axis_index("c")
        base = wid * B_PER_W
        pltpu.async_copy(idx_hbm.at[pl.ds(base, B_PER_W)], idx_v,
                         sem_i).wait()
        pltpu.async_copy(table_hbm.at[idx_v], rows_v, sem_g).wait()
        pltpu.async_copy(rows_v, out_hbm.at[pl.ds(base, B_PER_W)],
                         sem_o).wait()

    return gather_kernel


_gather = _build()


@jax.jit
def kernel(data, ratio_target_lookup, targets):
    del data
    return _gather(ratio_target_lookup, targets)


# speedup vs baseline: 1.1569x; 1.0393x over previous
import functools

import jax
import jax.numpy as jnp
from jax import lax
from jax.experimental import pallas as pl
from jax.experimental.pallas import tpu as pltpu
from jax.experimental.pallas import tpu_sc as plsc

BATCH = 16384
NUM_CORES = 1
NUM_SUBCORES = 16
NUM_WORKERS = NUM_CORES * NUM_SUBCORES
B_PER_W = BATCH // NUM_WORKERS
CHUNK = 128
NCHUNK = B_PER_W // CHUNK


def _build():
    mesh = plsc.VectorSubcoreMesh(core_axis_name="c", subcore_axis_name="s", num_cores=1)

    @functools.partial(
        pl.kernel,
        mesh=mesh,
        out_type=jax.ShapeDtypeStruct((BATCH,), jnp.float32),
        scratch_types=[
            pltpu.VMEM((B_PER_W,), jnp.int32),
            pltpu.VMEM((B_PER_W,), jnp.float32),
            pltpu.SemaphoreType.DMA,
            pltpu.SemaphoreType.DMA,
            pltpu.SemaphoreType.DMA,
        ],
    )
    def gather_kernel(table_hbm, idx_hbm, out_hbm, idx_v, rows_v,
                      sem_i, sem_g, sem_o):
        wid = lax.axis_index("s") * NUM_CORES + lax.axis_index("c")
        base = wid * B_PER_W
        pltpu.async_copy(idx_hbm.at[pl.ds(base, B_PER_W)], idx_v,
                         sem_i).wait()
        pltpu.async_copy(table_hbm.at[idx_v], rows_v, sem_g).wait()
        pltpu.async_copy(rows_v, out_hbm.at[pl.ds(base, B_PER_W)],
                         sem_o).wait()

    return gather_kernel


_gather = _build()


@jax.jit
def kernel(data, ratio_target_lookup, targets):
    del data
    return _gather(ratio_target_lookup, targets)
